# Initial kernel scaffold; baseline (speedup 1.0000x reference)
import jax
import jax.numpy as jnp
from jax.experimental import pallas as pl


def kernel(x, edge_index, W1, b1, W2, b2, W3, b3, Wc, bc):
    raise NotImplementedError("write your pallas kernel here")


# speedup vs baseline: 24.7092x; 24.7092x over previous
import functools

import jax
import jax.numpy as jnp
from jax import lax
from jax.experimental import pallas as pl
from jax.experimental.pallas import tpu as pltpu
from jax.experimental.pallas import tpu_sc as plsc

NC = 2
NS = 16
CH = 125


def _sc_mesh():
    return plsc.VectorSubcoreMesh(core_axis_name="c", subcore_axis_name="s")


def _tile_row_copy(s, n, src_at, dst_at):
    base = (n // NS) // 8 * 8
    rem = n - NS * base
    pltpu.sync_copy(src_at(s * base, base), dst_at(s * base, base))
    if rem:
        @pl.when(s == NS - 1)
        def _():
            pltpu.sync_copy(src_at(NS * base, rem), dst_at(NS * base, rem))


@functools.lru_cache(maxsize=None)
def _make_deg_kernel(n, e):
    ept = e // (NC * NS)
    nch = ept // CH

    @functools.partial(
        pl.kernel,
        mesh=_sc_mesh(),
        compiler_params=pltpu.CompilerParams(use_tc_tiling_on_sc=False),
        out_type=jax.ShapeDtypeStruct((NC, n, 16), jnp.float32),
        scratch_types=[
            pltpu.VMEM((nch, CH), jnp.int32),
            pltpu.VMEM((CH, 16), jnp.float32),
            pltpu.VMEM_SHARED((n, 16), jnp.float32),
        ],
    )
    def deg_kernel(dst_hbm, ones_hbm, zero_hbm, out_hbm, dst_v, ones_v, hist_sh):
        c = lax.axis_index("c")
        s = lax.axis_index("s")
        rowbase = (c * NS + s) * nch
        pltpu.sync_copy(dst_hbm.at[pl.ds(rowbase, nch)], dst_v)
        pltpu.sync_copy(ones_hbm, ones_v)
        _tile_row_copy(s, n,
                       lambda o, l: zero_hbm.at[pl.ds(o, l)],
                       lambda o, l: hist_sh.at[pl.ds(o, l)])
        plsc.subcore_barrier()

        def body(j, carry):
            pltpu.sync_copy(ones_v, hist_sh.at[dst_v.at[j]], add=True)
            return carry

        lax.fori_loop(0, nch, body, 0)
        plsc.subcore_barrier()
        _tile_row_copy(s, n,
                       lambda o, l: hist_sh.at[pl.ds(o, l)],
                       lambda o, l: out_hbm.at[c, pl.ds(o, l)])

    return deg_kernel


def _gather_scatter_loop(nch, mp_hbm, src_v, dst_v, rows_v, agg_sh, sem):
    pltpu.async_copy(mp_hbm.at[src_v.at[0]], rows_v.at[0], sem)

    def body(j, carry):
        buf = lax.rem(j, 2)
        nbuf = lax.rem(j + 1, 2)
        pltpu.make_async_copy(mp_hbm.at[src_v.at[j]], rows_v.at[buf],
                              sem).wait()

        @pl.when(j + 1 < nch)
        def _():
            pltpu.async_copy(mp_hbm.at[src_v.at[j + 1]], rows_v.at[nbuf], sem)

        pltpu.sync_copy(rows_v.at[buf], agg_sh.at[dst_v.at[j]], add=True)
        return carry

    lax.fori_loop(0, nch, body, 0)


@functools.lru_cache(maxsize=None)
def _make_agg_kernel(n, e, f):
    ept = e // (NC * NS)
    nch = ept // CH

    @functools.partial(
        pl.kernel,
        mesh=_sc_mesh(),
        compiler_params=pltpu.CompilerParams(use_tc_tiling_on_sc=False),
        out_type=jax.ShapeDtypeStruct((NC, n, f), jnp.float32),
        scratch_types=[
            pltpu.VMEM((nch, CH), jnp.int32),
            pltpu.VMEM((nch, CH), jnp.int32),
            pltpu.VMEM((2, CH, f), jnp.float32),
            pltpu.VMEM_SHARED((n, f), jnp.float32),
            pltpu.SemaphoreType.DMA,
        ],
    )
    def agg_kernel(src_hbm, dst_hbm, mp_hbm, zero_hbm, out_hbm,
                   src_v, dst_v, rows_v, agg_sh, sem):
        c = lax.axis_index("c")
        s = lax.axis_index("s")
        rowbase = (c * NS + s) * nch
        pltpu.sync_copy(src_hbm.at[pl.ds(rowbase, nch)], src_v)
        pltpu.sync_copy(dst_hbm.at[pl.ds(rowbase, nch)], dst_v)
        _tile_row_copy(s, n,
                       lambda o, l: zero_hbm.at[pl.ds(o, l)],
                       lambda o, l: agg_sh.at[pl.ds(o, l)])
        plsc.subcore_barrier()
        _gather_scatter_loop(nch, mp_hbm, src_v, dst_v, rows_v, agg_sh, sem)
        plsc.subcore_barrier()
        _tile_row_copy(s, n,
                       lambda o, l: agg_sh.at[pl.ds(o, l)],
                       lambda o, l: out_hbm.at[c, pl.ds(o, l)])

    return agg_kernel


@functools.lru_cache(maxsize=None)
def _make_agg_split_kernel(n, e, half):
    ept = e // NS
    nch = ept // CH

    @functools.partial(
        pl.kernel,
        mesh=_sc_mesh(),
        compiler_params=pltpu.CompilerParams(use_tc_tiling_on_sc=False),
        out_type=jax.ShapeDtypeStruct((NC * n, half), jnp.float32),
        scratch_types=[
            pltpu.VMEM((nch, CH), jnp.int32),
            pltpu.VMEM((nch, CH), jnp.int32),
            pltpu.VMEM((2, CH, half), jnp.float32),
            pltpu.VMEM_SHARED((n, half), jnp.float32),
            pltpu.SemaphoreType.DMA,
        ],
    )
    def agg_kernel(src_hbm, srcp_hbm, dst_hbm, mp_hbm, zero_hbm, out_hbm,
                   src_v, dst_v, rows_v, agg_sh, sem):
        c = lax.axis_index("c")
        s = lax.axis_index("s")
        rowbase = s * nch

        @pl.when(c == 0)
        def _():
            pltpu.sync_copy(src_hbm.at[pl.ds(rowbase, nch)], src_v)

        @pl.when(c == 1)
        def _():
            pltpu.sync_copy(srcp_hbm.at[pl.ds(rowbase, nch)], src_v)

        pltpu.sync_copy(dst_hbm.at[pl.ds(rowbase, nch)], dst_v)
        _tile_row_copy(s, n,
                       lambda o, l: zero_hbm.at[pl.ds(o, l)],
                       lambda o, l: agg_sh.at[pl.ds(o, l)])
        plsc.subcore_barrier()
        _gather_scatter_loop(nch, mp_hbm, src_v, dst_v, rows_v, agg_sh, sem)
        plsc.subcore_barrier()
        _tile_row_copy(s, n,
                       lambda o, l: agg_sh.at[pl.ds(o, l)],
                       lambda o, l: out_hbm.at[pl.ds(c * n + o, l)])

    return agg_kernel


def _prep_body(degh_ref, x_ref, w_ref, dinv_ref, mp_ref):
    half = mp_ref.shape[2]
    deg = degh_ref[0, :, 0] + degh_ref[1, :, 0] + 1.0
    dinv = lax.rsqrt(deg)[:, None]
    dinv_ref[...] = dinv
    m = dinv * jnp.dot(x_ref[...], w_ref[...])
    mp_ref[0] = m[:, :half]
    mp_ref[1] = m[:, half:]


def _mid_split_body(slo_ref, shi_ref, mplo_ref, mphi_ref, dinv_ref, b_ref,
                    w_ref, mn_ref):
    dinv = dinv_ref[...]
    agg = jnp.concatenate([slo_ref[...] + mplo_ref[...],
                           shi_ref[...] + mphi_ref[...]], axis=-1)
    h = jnp.tanh(dinv * agg + b_ref[...])
    mn_ref[...] = dinv * jnp.dot(h, w_ref[...])


def _mid_body(s_ref, mp_ref, dinv_ref, b_ref, w_ref, mn_ref):
    dinv = dinv_ref[...]
    h = jnp.tanh(dinv * (s_ref[0] + s_ref[1] + mp_ref[...]) + b_ref[...])
    mn_ref[...] = dinv * jnp.dot(h, w_ref[...])


def _final_body(s_ref, mp_ref, dinv_ref, b_ref, wc_ref, bc_ref,
                out_ref, h_ref):
    h = jnp.tanh(dinv_ref[...] * (s_ref[0] + s_ref[1] + mp_ref[...])
                 + b_ref[...])
    h_ref[...] = h
    out_ref[...] = jnp.dot(h, wc_ref[...]) + bc_ref[...]


def kernel(x, edge_index, W1, b1, W2, b2, W3, b3, Wc, bc):
    n, d_in = x.shape
    e = edge_index.shape[1]
    f1, f2, f3 = W1.shape[1], W2.shape[1], W3.shape[1]
    half = f1 // 2
    ncls = Wc.shape[1]
    br = 2000
    grid = (n // br,)
    nb = n // br

    src2d = edge_index[0].reshape(e // CH, CH)
    srcp2d = src2d + n
    dst2d = edge_index[1].reshape(e // CH, CH)
    ones16 = jnp.ones((CH, 16), jnp.float32)
    zeros = {f: jnp.zeros((n, f), jnp.float32) for f in {16, half, f2, f3}}

    degh = _make_deg_kernel(n, e)(dst2d, ones16, zeros[16])

    rows = lambda i: (i, 0)
    rows_hi = lambda i: (i + nb, 0)
    fixed = lambda i: (0, 0)
    part = lambda i: (0, i, 0)

    dinv, mp1 = pl.pallas_call(
        _prep_body,
        grid=grid,
        in_specs=[
            pl.BlockSpec((NC, br, 16), part),
            pl.BlockSpec((br, d_in), rows),
            pl.BlockSpec((d_in, f1), fixed),
        ],
        out_specs=[
            pl.BlockSpec((br, 1), rows),
            pl.BlockSpec((2, br, half), part),
        ],
        out_shape=[
            jax.ShapeDtypeStruct((n, 1), jnp.float32),
            jax.ShapeDtypeStruct((2, n, half), jnp.float32),
        ],
    )(degh, x, W1)
    mp1f = mp1.reshape(2 * n, half)

    s1f = _make_agg_split_kernel(n, e, half)(
        src2d, srcp2d, dst2d, mp1f, zeros[half])

    mp2 = pl.pallas_call(
        _mid_split_body,
        grid=grid,
        in_specs=[
            pl.BlockSpec((br, half), rows),
            pl.BlockSpec((br, half), rows_hi),
            pl.BlockSpec((br, half), rows),
            pl.BlockSpec((br, half), rows_hi),
            pl.BlockSpec((br, 1), rows),
            pl.BlockSpec((1, f1), fixed),
            pl.BlockSpec((f1, f2), fixed),
        ],
        out_specs=pl.BlockSpec((br, f2), rows),
        out_shape=jax.ShapeDtypeStruct((n, f2), jnp.float32),
    )(s1f, s1f, mp1f, mp1f, dinv, b1.reshape(1, f1), W2)

    s2 = _make_agg_kernel(n, e, f2)(src2d, dst2d, mp2, zeros[f2])

    mp3 = pl.pallas_call(
        _mid_body,
        grid=grid,
        in_specs=[
            pl.BlockSpec((NC, br, f2), part),
            pl.BlockSpec((br, f2), rows),
            pl.BlockSpec((br, 1), rows),
            pl.BlockSpec((1, f2), fixed),
            pl.BlockSpec((f2, f3), fixed),
        ],
        out_specs=pl.BlockSpec((br, f3), rows),
        out_shape=jax.ShapeDtypeStruct((n, f3), jnp.float32),
    )(s2, mp2, dinv, b2.reshape(1, f2), W3)

    s3 = _make_agg_kernel(n, e, f3)(src2d, dst2d, mp3, zeros[f3])

    out, h3 = pl.pallas_call(
        _final_body,
        grid=grid,
        in_specs=[
            pl.BlockSpec((NC, br, f3), part),
            pl.BlockSpec((br, f3), rows),
            pl.BlockSpec((br, 1), rows),
            pl.BlockSpec((1, f3), fixed),
            pl.BlockSpec((f3, ncls), fixed),
            pl.BlockSpec((1, ncls), fixed),
        ],
        out_specs=[
            pl.BlockSpec((br, ncls), rows),
            pl.BlockSpec((br, f3), rows),
        ],
        out_shape=[
            jax.ShapeDtypeStruct((n, ncls), jnp.float32),
            jax.ShapeDtypeStruct((n, f3), jnp.float32),
        ],
    )(s3, mp3, dinv, b3.reshape(1, f3), Wc, bc.reshape(1, ncls))

    return (out, h3)


# speedup vs baseline: 36.5002x; 1.4772x over previous
import functools

import jax
import jax.numpy as jnp
from jax import lax
from jax.experimental import pallas as pl
from jax.experimental.pallas import tpu as pltpu
from jax.experimental.pallas import tpu_sc as plsc

NC = 2
NS = 16
CH = 125


def _sc_mesh():
    return plsc.VectorSubcoreMesh(core_axis_name="c", subcore_axis_name="s")


def _tile_row_copy(s, n, src_at, dst_at):
    base = (n // NS) // 8 * 8
    rem = n - NS * base
    pltpu.sync_copy(src_at(s * base, base), dst_at(s * base, base))
    if rem:
        @pl.when(s == NS - 1)
        def _():
            pltpu.sync_copy(src_at(NS * base, rem), dst_at(NS * base, rem))


@functools.lru_cache(maxsize=None)
def _make_deg_kernel(n, e):
    ept = e // (NC * NS)
    nch = ept // CH

    @functools.partial(
        pl.kernel,
        mesh=_sc_mesh(),
        compiler_params=pltpu.CompilerParams(use_tc_tiling_on_sc=False),
        out_type=jax.ShapeDtypeStruct((NC, n, 16), jnp.float32),
        scratch_types=[
            pltpu.VMEM((nch, CH), jnp.int32),
            pltpu.VMEM((CH, 16), jnp.float32),
            pltpu.VMEM_SHARED((n, 16), jnp.float32),
            pltpu.SemaphoreType.DMA,
        ],
    )
    def deg_kernel(dst_hbm, ones_hbm, zero_hbm, out_hbm, dst_v, ones_v,
                   hist_sh, sem):
        c = lax.axis_index("c")
        s = lax.axis_index("s")
        rowbase = (c * NS + s) * nch
        pltpu.sync_copy(dst_hbm.at[pl.ds(rowbase, nch)], dst_v)
        pltpu.sync_copy(ones_hbm, ones_v)
        _tile_row_copy(s, n,
                       lambda o, l: zero_hbm.at[pl.ds(o, l)],
                       lambda o, l: hist_sh.at[pl.ds(o, l)])
        plsc.subcore_barrier()

        def body(j, carry):
            pltpu.async_copy(ones_v, hist_sh.at[dst_v.at[j]], sem, add=True)
            return carry

        lax.fori_loop(0, nch, body, 0)

        def drain(j, carry):
            pltpu.make_async_copy(ones_v, hist_sh.at[dst_v.at[j]], sem).wait()
            return carry

        lax.fori_loop(0, nch, drain, 0)
        plsc.subcore_barrier()
        _tile_row_copy(s, n,
                       lambda o, l: hist_sh.at[pl.ds(o, l)],
                       lambda o, l: out_hbm.at[c, pl.ds(o, l)])

    return deg_kernel


def _gather_scatter_loop(nch, nb, ga, mp_hbm, src_v, dst_v, rows_v, agg_sh,
                         gsem, ssem):
    ns = nb - ga
    for k in range(ga):
        pltpu.async_copy(mp_hbm.at[src_v.at[k]], rows_v.at[k], gsem)

    def body(j, carry):
        @pl.when(j >= ns)
        def _():
            pltpu.make_async_copy(rows_v.at[lax.rem(j - ns, nb)],
                                  agg_sh.at[dst_v.at[j - ns]], ssem).wait()

        @pl.when(j + ga < nch)
        def _():
            pltpu.async_copy(mp_hbm.at[src_v.at[j + ga]],
                             rows_v.at[lax.rem(j + ga, nb)], gsem)

        pltpu.make_async_copy(mp_hbm.at[src_v.at[j]],
                              rows_v.at[lax.rem(j, nb)], gsem).wait()
        pltpu.async_copy(rows_v.at[lax.rem(j, nb)],
                         agg_sh.at[dst_v.at[j]], ssem, add=True)
        return carry

    lax.fori_loop(0, nch, body, 0)
    for k in range(nch - ns, nch):
        pltpu.make_async_copy(rows_v.at[k % nb],
                              agg_sh.at[dst_v.at[k]], ssem).wait()


@functools.lru_cache(maxsize=None)
def _make_agg_kernel(n, e, f, nb, ga):
    ept = e // (NC * NS)
    nch = ept // CH

    @functools.partial(
        pl.kernel,
        mesh=_sc_mesh(),
        compiler_params=pltpu.CompilerParams(use_tc_tiling_on_sc=False),
        out_type=jax.ShapeDtypeStruct((NC, n, f), jnp.float32),
        scratch_types=[
            pltpu.VMEM((nch, CH), jnp.int32),
            pltpu.VMEM((nch, CH), jnp.int32),
            pltpu.VMEM((nb, CH, f), jnp.float32),
            pltpu.VMEM_SHARED((n, f), jnp.float32),
            pltpu.SemaphoreType.DMA,
            pltpu.SemaphoreType.DMA,
        ],
    )
    def agg_kernel(src_hbm, dst_hbm, mp_hbm, zero_hbm, out_hbm,
                   src_v, dst_v, rows_v, agg_sh, gsem, ssem):
        c = lax.axis_index("c")
        s = lax.axis_index("s")
        rowbase = (c * NS + s) * nch
        pltpu.sync_copy(src_hbm.at[pl.ds(rowbase, nch)], src_v)
        pltpu.sync_copy(dst_hbm.at[pl.ds(rowbase, nch)], dst_v)
        _tile_row_copy(s, n,
                       lambda o, l: zero_hbm.at[pl.ds(o, l)],
                       lambda o, l: agg_sh.at[pl.ds(o, l)])
        plsc.subcore_barrier()
        _gather_scatter_loop(nch, nb, ga, mp_hbm, src_v, dst_v, rows_v,
                             agg_sh, gsem, ssem)
        plsc.subcore_barrier()
        _tile_row_copy(s, n,
                       lambda o, l: agg_sh.at[pl.ds(o, l)],
                       lambda o, l: out_hbm.at[c, pl.ds(o, l)])

    return agg_kernel


@functools.lru_cache(maxsize=None)
def _make_agg_split_kernel(n, e, half, nb, ga):
    ept = e // NS
    nch = ept // CH

    @functools.partial(
        pl.kernel,
        mesh=_sc_mesh(),
        compiler_params=pltpu.CompilerParams(use_tc_tiling_on_sc=False),
        out_type=jax.ShapeDtypeStruct((NC * n, half), jnp.float32),
        scratch_types=[
            pltpu.VMEM((nch, CH), jnp.int32),
            pltpu.VMEM((nch, CH), jnp.int32),
            pltpu.VMEM((nb, CH, half), jnp.float32),
            pltpu.VMEM_SHARED((n, half), jnp.float32),
            pltpu.SemaphoreType.DMA,
            pltpu.SemaphoreType.DMA,
        ],
    )
    def agg_kernel(src_hbm, srcp_hbm, dst_hbm, mp_hbm, zero_hbm, out_hbm,
                   src_v, dst_v, rows_v, agg_sh, gsem, ssem):
        c = lax.axis_index("c")
        s = lax.axis_index("s")
        rowbase = s * nch

        @pl.when(c == 0)
        def _():
            pltpu.sync_copy(src_hbm.at[pl.ds(rowbase, nch)], src_v)

        @pl.when(c == 1)
        def _():
            pltpu.sync_copy(srcp_hbm.at[pl.ds(rowbase, nch)], src_v)

        pltpu.sync_copy(dst_hbm.at[pl.ds(rowbase, nch)], dst_v)
        _tile_row_copy(s, n,
                       lambda o, l: zero_hbm.at[pl.ds(o, l)],
                       lambda o, l: agg_sh.at[pl.ds(o, l)])
        plsc.subcore_barrier()
        _gather_scatter_loop(nch, nb, ga, mp_hbm, src_v, dst_v, rows_v,
                             agg_sh, gsem, ssem)
        plsc.subcore_barrier()
        _tile_row_copy(s, n,
                       lambda o, l: agg_sh.at[pl.ds(o, l)],
                       lambda o, l: out_hbm.at[pl.ds(c * n + o, l)])

    return agg_kernel


def _prep_body(degh_ref, x_ref, w_ref, dinv_ref, mp_ref):
    half = mp_ref.shape[2]
    deg = degh_ref[0, :, 0] + degh_ref[1, :, 0] + 1.0
    dinv = lax.rsqrt(deg)[:, None]
    dinv_ref[...] = dinv
    m = dinv * jnp.dot(x_ref[...], w_ref[...])
    mp_ref[0] = m[:, :half]
    mp_ref[1] = m[:, half:]


def _mid_split_body(slo_ref, shi_ref, mplo_ref, mphi_ref, dinv_ref, b_ref,
                    w_ref, mn_ref):
    dinv = dinv_ref[...]
    agg = jnp.concatenate([slo_ref[...] + mplo_ref[...],
                           shi_ref[...] + mphi_ref[...]], axis=-1)
    h = jnp.tanh(dinv * agg + b_ref[...])
    mn_ref[...] = dinv * jnp.dot(h, w_ref[...])


def _mid_body(s_ref, mp_ref, dinv_ref, b_ref, w_ref, mn_ref):
    dinv = dinv_ref[...]
    h = jnp.tanh(dinv * (s_ref[0] + s_ref[1] + mp_ref[...]) + b_ref[...])
    mn_ref[...] = dinv * jnp.dot(h, w_ref[...])


def _final_body(s_ref, mp_ref, dinv_ref, b_ref, wc_ref, bc_ref,
                out_ref, h_ref):
    h = jnp.tanh(dinv_ref[...] * (s_ref[0] + s_ref[1] + mp_ref[...])
                 + b_ref[...])
    h_ref[...] = h
    out_ref[...] = jnp.dot(h, wc_ref[...]) + bc_ref[...]


def kernel(x, edge_index, W1, b1, W2, b2, W3, b3, Wc, bc):
    n, d_in = x.shape
    e = edge_index.shape[1]
    f1, f2, f3 = W1.shape[1], W2.shape[1], W3.shape[1]
    half = f1 // 2
    ncls = Wc.shape[1]
    br = 2000
    grid = (n // br,)
    nb = n // br

    src2d = edge_index[0].reshape(e // CH, CH)
    srcp2d = src2d + n
    dst2d = edge_index[1].reshape(e // CH, CH)
    ones16 = jnp.ones((CH, 16), jnp.float32)
    zeros = {f: jnp.zeros((n, f), jnp.float32) for f in {16, half, f2, f3}}

    degh = _make_deg_kernel(n, e)(dst2d, ones16, zeros[16])

    rows = lambda i: (i, 0)
    rows_hi = lambda i: (i + nb, 0)
    fixed = lambda i: (0, 0)
    part = lambda i: (0, i, 0)

    dinv, mp1 = pl.pallas_call(
        _prep_body,
        grid=grid,
        in_specs=[
            pl.BlockSpec((NC, br, 16), part),
            pl.BlockSpec((br, d_in), rows),
            pl.BlockSpec((d_in, f1), fixed),
        ],
        out_specs=[
            pl.BlockSpec((br, 1), rows),
            pl.BlockSpec((2, br, half), part),
        ],
        out_shape=[
            jax.ShapeDtypeStruct((n, 1), jnp.float32),
            jax.ShapeDtypeStruct((2, n, half), jnp.float32),
        ],
    )(degh, x, W1)
    mp1f = mp1.reshape(2 * n, half)

    s1f = _make_agg_split_kernel(n, e, half, 6, 3)(
        src2d, srcp2d, dst2d, mp1f, zeros[half])

    mp2 = pl.pallas_call(
        _mid_split_body,
        grid=grid,
        in_specs=[
            pl.BlockSpec((br, half), rows),
            pl.BlockSpec((br, half), rows_hi),
            pl.BlockSpec((br, half), rows),
            pl.BlockSpec((br, half), rows_hi),
            pl.BlockSpec((br, 1), rows),
            pl.BlockSpec((1, f1), fixed),
            pl.BlockSpec((f1, f2), fixed),
        ],
        out_specs=pl.BlockSpec((br, f2), rows),
        out_shape=jax.ShapeDtypeStruct((n, f2), jnp.float32),
    )(s1f, s1f, mp1f, mp1f, dinv, b1.reshape(1, f1), W2)

    s2 = _make_agg_kernel(n, e, f2, 8, 4)(src2d, dst2d, mp2, zeros[f2])

    mp3 = pl.pallas_call(
        _mid_body,
        grid=grid,
        in_specs=[
            pl.BlockSpec((NC, br, f2), part),
            pl.BlockSpec((br, f2), rows),
            pl.BlockSpec((br, 1), rows),
            pl.BlockSpec((1, f2), fixed),
            pl.BlockSpec((f2, f3), fixed),
        ],
        out_specs=pl.BlockSpec((br, f3), rows),
        out_shape=jax.ShapeDtypeStruct((n, f3), jnp.float32),
    )(s2, mp2, dinv, b2.reshape(1, f2), W3)

    s3 = _make_agg_kernel(n, e, f3, 8, 4)(src2d, dst2d, mp3, zeros[f3])

    out, h3 = pl.pallas_call(
        _final_body,
        grid=grid,
        in_specs=[
            pl.BlockSpec((NC, br, f3), part),
            pl.BlockSpec((br, f3), rows),
            pl.BlockSpec((br, 1), rows),
            pl.BlockSpec((1, f3), fixed),
            pl.BlockSpec((f3, ncls), fixed),
            pl.BlockSpec((1, ncls), fixed),
        ],
        out_specs=[
            pl.BlockSpec((br, ncls), rows),
            pl.BlockSpec((br, f3), rows),
        ],
        out_shape=[
            jax.ShapeDtypeStruct((n, ncls), jnp.float32),
            jax.ShapeDtypeStruct((n, f3), jnp.float32),
        ],
    )(s3, mp3, dinv, b3.reshape(1, f3), Wc, bc.reshape(1, ncls))

    return (out, h3)


# speedup vs baseline: 37.3598x; 1.0235x over previous
import functools

import jax
import jax.numpy as jnp
from jax import lax
from jax.experimental import pallas as pl
from jax.experimental.pallas import tpu as pltpu
from jax.experimental.pallas import tpu_sc as plsc

NC = 2
NS = 16
CH = 125


def _sc_mesh():
    return plsc.VectorSubcoreMesh(core_axis_name="c", subcore_axis_name="s")


def _tile_row_copy(s, n, src_at, dst_at):
    base = (n // NS) // 8 * 8
    rem = n - NS * base
    pltpu.sync_copy(src_at(s * base, base), dst_at(s * base, base))
    if rem:
        @pl.when(s == NS - 1)
        def _():
            pltpu.sync_copy(src_at(NS * base, rem), dst_at(NS * base, rem))


@functools.lru_cache(maxsize=None)
def _make_deg_kernel(n, e):
    ept = e // (NC * NS)
    nch = ept // CH

    @functools.partial(
        pl.kernel,
        mesh=_sc_mesh(),
        compiler_params=pltpu.CompilerParams(use_tc_tiling_on_sc=False),
        out_type=jax.ShapeDtypeStruct((NC, n, 16), jnp.float32),
        scratch_types=[
            pltpu.VMEM((nch, CH), jnp.int32),
            pltpu.VMEM((CH, 16), jnp.float32),
            pltpu.VMEM_SHARED((n, 16), jnp.float32),
            pltpu.SemaphoreType.DMA,
        ],
    )
    def deg_kernel(dst_hbm, ones_hbm, zero_hbm, out_hbm, dst_v, ones_v,
                   hist_sh, sem):
        c = lax.axis_index("c")
        s = lax.axis_index("s")
        rowbase = (c * NS + s) * nch
        pltpu.sync_copy(dst_hbm.at[pl.ds(rowbase, nch)], dst_v)
        pltpu.sync_copy(ones_hbm, ones_v)
        _tile_row_copy(s, n,
                       lambda o, l: zero_hbm.at[pl.ds(o, l)],
                       lambda o, l: hist_sh.at[pl.ds(o, l)])
        plsc.subcore_barrier()

        def body(j, carry):
            pltpu.async_copy(ones_v, hist_sh.at[dst_v.at[j]], sem, add=True)
            return carry

        lax.fori_loop(0, nch, body, 0)

        def drain(j, carry):
            pltpu.make_async_copy(ones_v, hist_sh.at[dst_v.at[j]], sem).wait()
            return carry

        lax.fori_loop(0, nch, drain, 0)
        plsc.subcore_barrier()
        _tile_row_copy(s, n,
                       lambda o, l: hist_sh.at[pl.ds(o, l)],
                       lambda o, l: out_hbm.at[c, pl.ds(o, l)])

    return deg_kernel


def _gather_scatter_loop(nch, nb, ga, mp_hbm, src_v, dst_v, rows_v, agg_sh,
                         gsem, ssem):
    ns = nb - ga
    for k in range(ga):
        pltpu.async_copy(mp_hbm.at[src_v.at[k]], rows_v.at[k], gsem)

    def body(j, carry):
        @pl.when(j >= ns)
        def _():
            pltpu.make_async_copy(rows_v.at[lax.rem(j - ns, nb)],
                                  agg_sh.at[dst_v.at[j - ns]], ssem).wait()

        @pl.when(j + ga < nch)
        def _():
            pltpu.async_copy(mp_hbm.at[src_v.at[j + ga]],
                             rows_v.at[lax.rem(j + ga, nb)], gsem)

        pltpu.make_async_copy(mp_hbm.at[src_v.at[j]],
                              rows_v.at[lax.rem(j, nb)], gsem).wait()
        pltpu.async_copy(rows_v.at[lax.rem(j, nb)],
                         agg_sh.at[dst_v.at[j]], ssem, add=True)
        return carry

    lax.fori_loop(0, nch, body, 0)
    for k in range(nch - ns, nch):
        pltpu.make_async_copy(rows_v.at[k % nb],
                              agg_sh.at[dst_v.at[k]], ssem).wait()


@functools.lru_cache(maxsize=None)
def _make_agg_kernel(n, e, f, nb, ga):
    ept = e // (NC * NS)
    nch = ept // CH

    @functools.partial(
        pl.kernel,
        mesh=_sc_mesh(),
        compiler_params=pltpu.CompilerParams(use_tc_tiling_on_sc=False),
        out_type=jax.ShapeDtypeStruct((NC, n, f), jnp.float32),
        scratch_types=[
            pltpu.VMEM((nch, CH), jnp.int32),
            pltpu.VMEM((nch, CH), jnp.int32),
            pltpu.VMEM((nb, CH, f), jnp.float32),
            pltpu.VMEM_SHARED((n, f), jnp.float32),
            pltpu.SemaphoreType.DMA,
            pltpu.SemaphoreType.DMA,
        ],
    )
    def agg_kernel(src_hbm, dst_hbm, mp_hbm, zero_hbm, out_hbm,
                   src_v, dst_v, rows_v, agg_sh, gsem, ssem):
        c = lax.axis_index("c")
        s = lax.axis_index("s")
        rowbase = (c * NS + s) * nch
        pltpu.sync_copy(src_hbm.at[pl.ds(rowbase, nch)], src_v)
        pltpu.sync_copy(dst_hbm.at[pl.ds(rowbase, nch)], dst_v)
        _tile_row_copy(s, n,
                       lambda o, l: zero_hbm.at[pl.ds(o, l)],
                       lambda o, l: agg_sh.at[pl.ds(o, l)])
        plsc.subcore_barrier()
        _gather_scatter_loop(nch, nb, ga, mp_hbm, src_v, dst_v, rows_v,
                             agg_sh, gsem, ssem)
        plsc.subcore_barrier()
        _tile_row_copy(s, n,
                       lambda o, l: agg_sh.at[pl.ds(o, l)],
                       lambda o, l: out_hbm.at[c, pl.ds(o, l)])

    return agg_kernel


@functools.lru_cache(maxsize=None)
def _make_agg_split_kernel(n, e, half, nb, ga):
    ept = e // NS
    nch = ept // CH

    @functools.partial(
        pl.kernel,
        mesh=_sc_mesh(),
        compiler_params=pltpu.CompilerParams(use_tc_tiling_on_sc=False),
        out_type=jax.ShapeDtypeStruct((NC * n, half), jnp.float32),
        scratch_types=[
            pltpu.VMEM((nch, CH), jnp.int32),
            pltpu.VMEM((nch, CH), jnp.int32),
            pltpu.VMEM((nb, CH, half), jnp.float32),
            pltpu.VMEM_SHARED((n, half), jnp.float32),
            pltpu.SemaphoreType.DMA,
            pltpu.SemaphoreType.DMA,
        ],
    )
    def agg_kernel(src_hbm, srcp_hbm, dst_hbm, mp_hbm, zero_hbm, out_hbm,
                   src_v, dst_v, rows_v, agg_sh, gsem, ssem):
        c = lax.axis_index("c")
        s = lax.axis_index("s")
        rowbase = s * nch

        @pl.when(c == 0)
        def _():
            pltpu.sync_copy(src_hbm.at[pl.ds(rowbase, nch)], src_v)

        @pl.when(c == 1)
        def _():
            pltpu.sync_copy(srcp_hbm.at[pl.ds(rowbase, nch)], src_v)

        pltpu.sync_copy(dst_hbm.at[pl.ds(rowbase, nch)], dst_v)
        _tile_row_copy(s, n,
                       lambda o, l: zero_hbm.at[pl.ds(o, l)],
                       lambda o, l: agg_sh.at[pl.ds(o, l)])
        plsc.subcore_barrier()
        _gather_scatter_loop(nch, nb, ga, mp_hbm, src_v, dst_v, rows_v,
                             agg_sh, gsem, ssem)
        plsc.subcore_barrier()
        _tile_row_copy(s, n,
                       lambda o, l: agg_sh.at[pl.ds(o, l)],
                       lambda o, l: out_hbm.at[pl.ds(c * n + o, l)])

    return agg_kernel


def _prep_body(degh_ref, x_ref, w_ref, dinv_ref, mp_ref):
    half = mp_ref.shape[2]
    deg = degh_ref[0, :, 0] + degh_ref[1, :, 0] + 1.0
    dinv = lax.rsqrt(deg)[:, None]
    dinv_ref[...] = dinv
    m = dinv * jnp.dot(x_ref[...], w_ref[...])
    mp_ref[0] = m[:, :half]
    mp_ref[1] = m[:, half:]


def _mid_split_body(slo_ref, shi_ref, mplo_ref, mphi_ref, dinv_ref, b_ref,
                    w_ref, mn_ref):
    dinv = dinv_ref[...]
    agg = jnp.concatenate([slo_ref[...] + mplo_ref[...],
                           shi_ref[...] + mphi_ref[...]], axis=-1)
    h = jnp.tanh(dinv * agg + b_ref[...])
    mn_ref[...] = dinv * jnp.dot(h, w_ref[...])


def _mid_body(s_ref, mp_ref, dinv_ref, b_ref, w_ref, mn_ref):
    dinv = dinv_ref[...]
    h = jnp.tanh(dinv * (s_ref[0] + s_ref[1] + mp_ref[...]) + b_ref[...])
    mn_ref[...] = dinv * jnp.dot(h, w_ref[...])


def _final_body(s_ref, mp_ref, dinv_ref, b_ref, wc_ref, bc_ref,
                out_ref, h_ref):
    h = jnp.tanh(dinv_ref[...] * (s_ref[0] + s_ref[1] + mp_ref[...])
                 + b_ref[...])
    h_ref[...] = h
    out_ref[...] = jnp.dot(h, wc_ref[...]) + bc_ref[...]


def kernel(x, edge_index, W1, b1, W2, b2, W3, b3, Wc, bc):
    n, d_in = x.shape
    e = edge_index.shape[1]
    f1, f2, f3 = W1.shape[1], W2.shape[1], W3.shape[1]
    half = f1 // 2
    ncls = Wc.shape[1]
    br = 2000
    grid = (n // br,)
    nb = n // br

    src2d = edge_index[0].reshape(e // CH, CH)
    srcp2d = src2d + n
    dst2d = edge_index[1].reshape(e // CH, CH)
    ones16 = jnp.ones((CH, 16), jnp.float32)
    zeros = {f: jnp.zeros((n, f), jnp.float32) for f in {16, half, f2, f3}}

    degh = _make_deg_kernel(n, e)(dst2d, ones16, zeros[16])

    rows = lambda i: (i, 0)
    rows_hi = lambda i: (i + nb, 0)
    fixed = lambda i: (0, 0)
    part = lambda i: (0, i, 0)

    dinv, mp1 = pl.pallas_call(
        _prep_body,
        grid=grid,
        in_specs=[
            pl.BlockSpec((NC, br, 16), part),
            pl.BlockSpec((br, d_in), rows),
            pl.BlockSpec((d_in, f1), fixed),
        ],
        out_specs=[
            pl.BlockSpec((br, 1), rows),
            pl.BlockSpec((2, br, half), part),
        ],
        out_shape=[
            jax.ShapeDtypeStruct((n, 1), jnp.float32),
            jax.ShapeDtypeStruct((2, n, half), jnp.float32),
        ],
    )(degh, x, W1)
    mp1f = mp1.reshape(2 * n, half)

    s1f = _make_agg_split_kernel(n, e, half, 6, 3)(
        src2d, srcp2d, dst2d, mp1f, zeros[half])

    mp2 = pl.pallas_call(
        _mid_split_body,
        grid=grid,
        in_specs=[
            pl.BlockSpec((br, half), rows),
            pl.BlockSpec((br, half), rows_hi),
            pl.BlockSpec((br, half), rows),
            pl.BlockSpec((br, half), rows_hi),
            pl.BlockSpec((br, 1), rows),
            pl.BlockSpec((1, f1), fixed),
            pl.BlockSpec((f1, f2), fixed),
        ],
        out_specs=pl.BlockSpec((br, f2), rows),
        out_shape=jax.ShapeDtypeStruct((n, f2), jnp.float32),
    )(s1f, s1f, mp1f, mp1f, dinv, b1.reshape(1, f1), W2)

    s2 = _make_agg_kernel(n, e, f2, 8, 4)(src2d, dst2d, mp2, zeros[f2])

    mp3 = pl.pallas_call(
        _mid_body,
        grid=grid,
        in_specs=[
            pl.BlockSpec((NC, br, f2), part),
            pl.BlockSpec((br, f2), rows),
            pl.BlockSpec((br, 1), rows),
            pl.BlockSpec((1, f2), fixed),
            pl.BlockSpec((f2, f3), fixed),
        ],
        out_specs=pl.BlockSpec((br, f3), rows),
        out_shape=jax.ShapeDtypeStruct((n, f3), jnp.float32),
    )(s2, mp2, dinv, b2.reshape(1, f2), W3)

    s3 = _make_agg_kernel(n, e, f3, 16, 8)(src2d, dst2d, mp3, zeros[f3])

    out, h3 = pl.pallas_call(
        _final_body,
        grid=grid,
        in_specs=[
            pl.BlockSpec((NC, br, f3), part),
            pl.BlockSpec((br, f3), rows),
            pl.BlockSpec((br, 1), rows),
            pl.BlockSpec((1, f3), fixed),
            pl.BlockSpec((f3, ncls), fixed),
            pl.BlockSpec((1, ncls), fixed),
        ],
        out_specs=[
            pl.BlockSpec((br, ncls), rows),
            pl.BlockSpec((br, f3), rows),
        ],
        out_shape=[
            jax.ShapeDtypeStruct((n, ncls), jnp.float32),
            jax.ShapeDtypeStruct((n, f3), jnp.float32),
        ],
    )(s3, mp3, dinv, b3.reshape(1, f3), Wc, bc.reshape(1, ncls))

    return (out, h3)


# speedup vs baseline: 38.4927x; 1.0303x over previous
import functools

import jax
import jax.numpy as jnp
from jax import lax
from jax.experimental import pallas as pl
from jax.experimental.pallas import tpu as pltpu
from jax.experimental.pallas import tpu_sc as plsc

NC = 2
NS = 16
CH = 125


def _sc_mesh():
    return plsc.VectorSubcoreMesh(core_axis_name="c", subcore_axis_name="s")


def _tile_row_copy(s, n, src_at, dst_at):
    base = (n // NS) // 8 * 8
    rem = n - NS * base
    pltpu.sync_copy(src_at(s * base, base), dst_at(s * base, base))
    if rem:
        @pl.when(s == NS - 1)
        def _():
            pltpu.sync_copy(src_at(NS * base, rem), dst_at(NS * base, rem))


@functools.lru_cache(maxsize=None)
def _make_deg_kernel(n, e):
    ept = e // (NC * NS)
    nch = ept // CH

    @functools.partial(
        pl.kernel,
        mesh=_sc_mesh(),
        compiler_params=pltpu.CompilerParams(use_tc_tiling_on_sc=False),
        out_type=jax.ShapeDtypeStruct((NC, n, 16), jnp.float32),
        scratch_types=[
            pltpu.VMEM((nch, CH), jnp.int32),
            pltpu.VMEM((CH, 16), jnp.float32),
            pltpu.VMEM_SHARED((n, 16), jnp.float32),
            pltpu.SemaphoreType.DMA,
        ],
    )
    def deg_kernel(ei_hbm, ones_hbm, zero_hbm, out_hbm, dst_v, ones_v,
                   hist_sh, sem):
        c = lax.axis_index("c")
        s = lax.axis_index("s")
        rowbase = (c * NS + s) * nch
        pltpu.sync_copy(ei_hbm.at[1, pl.ds(rowbase, nch)], dst_v)
        pltpu.sync_copy(ones_hbm, ones_v)
        _tile_row_copy(s, n,
                       lambda o, l: zero_hbm.at[pl.ds(o, l)],
                       lambda o, l: hist_sh.at[pl.ds(o, l)])
        plsc.subcore_barrier()

        def body(j, carry):
            pltpu.async_copy(ones_v, hist_sh.at[dst_v.at[j]], sem, add=True)
            return carry

        lax.fori_loop(0, nch, body, 0)

        def drain(j, carry):
            pltpu.make_async_copy(ones_v, hist_sh.at[dst_v.at[j]], sem).wait()
            return carry

        lax.fori_loop(0, nch, drain, 0)
        plsc.subcore_barrier()
        _tile_row_copy(s, n,
                       lambda o, l: hist_sh.at[pl.ds(o, l)],
                       lambda o, l: out_hbm.at[c, pl.ds(o, l)])

    return deg_kernel


def _gather_scatter_loop(nch, nb, ga, mp_hbm, src_v, dst_v, rows_v, agg_sh,
                         gsem, ssem):
    ns = nb - ga
    for k in range(ga):
        pltpu.async_copy(mp_hbm.at[src_v.at[k]], rows_v.at[k], gsem)

    def body(j, carry):
        @pl.when(j >= ns)
        def _():
            pltpu.make_async_copy(rows_v.at[lax.rem(j - ns, nb)],
                                  agg_sh.at[dst_v.at[j - ns]], ssem).wait()

        @pl.when(j + ga < nch)
        def _():
            pltpu.async_copy(mp_hbm.at[src_v.at[j + ga]],
                             rows_v.at[lax.rem(j + ga, nb)], gsem)

        pltpu.make_async_copy(mp_hbm.at[src_v.at[j]],
                              rows_v.at[lax.rem(j, nb)], gsem).wait()
        pltpu.async_copy(rows_v.at[lax.rem(j, nb)],
                         agg_sh.at[dst_v.at[j]], ssem, add=True)
        return carry

    lax.fori_loop(0, nch, body, 0)
    for k in range(nch - ns, nch):
        pltpu.make_async_copy(rows_v.at[k % nb],
                              agg_sh.at[dst_v.at[k]], ssem).wait()


@functools.lru_cache(maxsize=None)
def _make_agg_kernel(n, e, f, nb, ga):
    ept = e // (NC * NS)
    nch = ept // CH

    @functools.partial(
        pl.kernel,
        mesh=_sc_mesh(),
        compiler_params=pltpu.CompilerParams(use_tc_tiling_on_sc=False),
        out_type=jax.ShapeDtypeStruct((NC, n, f), jnp.float32),
        scratch_types=[
            pltpu.VMEM((nch, CH), jnp.int32),
            pltpu.VMEM((nch, CH), jnp.int32),
            pltpu.VMEM((nb, CH, f), jnp.float32),
            pltpu.VMEM_SHARED((n, f), jnp.float32),
            pltpu.SemaphoreType.DMA,
            pltpu.SemaphoreType.DMA,
        ],
    )
    def agg_kernel(ei_hbm, mp_hbm, zero_hbm, out_hbm,
                   src_v, dst_v, rows_v, agg_sh, gsem, ssem):
        c = lax.axis_index("c")
        s = lax.axis_index("s")
        rowbase = (c * NS + s) * nch
        pltpu.sync_copy(ei_hbm.at[0, pl.ds(rowbase, nch)], src_v)
        pltpu.sync_copy(ei_hbm.at[1, pl.ds(rowbase, nch)], dst_v)
        _tile_row_copy(s, n,
                       lambda o, l: zero_hbm.at[pl.ds(o, l)],
                       lambda o, l: agg_sh.at[pl.ds(o, l)])
        plsc.subcore_barrier()
        _gather_scatter_loop(nch, nb, ga, mp_hbm, src_v, dst_v, rows_v,
                             agg_sh, gsem, ssem)
        plsc.subcore_barrier()
        _tile_row_copy(s, n,
                       lambda o, l: agg_sh.at[pl.ds(o, l)],
                       lambda o, l: out_hbm.at[c, pl.ds(o, l)])

    return agg_kernel


@functools.lru_cache(maxsize=None)
def _make_agg_split_kernel(n, e, half, nb, ga):
    ept = e // NS
    nch = ept // CH

    @functools.partial(
        pl.kernel,
        mesh=_sc_mesh(),
        compiler_params=pltpu.CompilerParams(use_tc_tiling_on_sc=False),
        out_type=jax.ShapeDtypeStruct((NC * n, half), jnp.float32),
        scratch_types=[
            pltpu.VMEM((nch, CH), jnp.int32),
            pltpu.VMEM((nch, CH), jnp.int32),
            pltpu.VMEM((nb, CH, half), jnp.float32),
            pltpu.VMEM_SHARED((n, half), jnp.float32),
            pltpu.SemaphoreType.DMA,
            pltpu.SemaphoreType.DMA,
        ],
    )
    def agg_kernel(ei_hbm, mplo_hbm, mphi_hbm, zero_hbm, out_hbm,
                   src_v, dst_v, rows_v, agg_sh, gsem, ssem):
        c = lax.axis_index("c")
        s = lax.axis_index("s")
        rowbase = s * nch
        pltpu.sync_copy(ei_hbm.at[0, pl.ds(rowbase, nch)], src_v)
        pltpu.sync_copy(ei_hbm.at[1, pl.ds(rowbase, nch)], dst_v)
        _tile_row_copy(s, n,
                       lambda o, l: zero_hbm.at[pl.ds(o, l)],
                       lambda o, l: agg_sh.at[pl.ds(o, l)])
        plsc.subcore_barrier()

        @pl.when(c == 0)
        def _():
            _gather_scatter_loop(nch, nb, ga, mplo_hbm, src_v, dst_v, rows_v,
                                 agg_sh, gsem, ssem)

        @pl.when(c == 1)
        def _():
            _gather_scatter_loop(nch, nb, ga, mphi_hbm, src_v, dst_v, rows_v,
                                 agg_sh, gsem, ssem)

        plsc.subcore_barrier()
        _tile_row_copy(s, n,
                       lambda o, l: agg_sh.at[pl.ds(o, l)],
                       lambda o, l: out_hbm.at[pl.ds(c * n + o, l)])

    return agg_kernel


def _prep_body(degh_ref, x_ref, w_ref, dinv_ref, mplo_ref, mphi_ref):
    half = mplo_ref.shape[1]
    deg = degh_ref[0, :, 0] + degh_ref[1, :, 0] + 1.0
    dinv = lax.rsqrt(deg)[:, None]
    dinv_ref[...] = dinv
    m = dinv * jnp.dot(x_ref[...], w_ref[...])
    mplo_ref[...] = m[:, :half]
    mphi_ref[...] = m[:, half:]


def _mid_split_body(slo_ref, shi_ref, mplo_ref, mphi_ref, dinv_ref, b_ref,
                    w_ref, mn_ref):
    dinv = dinv_ref[...]
    agg = jnp.concatenate([slo_ref[...] + mplo_ref[...],
                           shi_ref[...] + mphi_ref[...]], axis=-1)
    h = jnp.tanh(dinv * agg + b_ref[...])
    mn_ref[...] = dinv * jnp.dot(h, w_ref[...])


def _mid_body(s_ref, mp_ref, dinv_ref, b_ref, w_ref, mn_ref):
    dinv = dinv_ref[...]
    h = jnp.tanh(dinv * (s_ref[0] + s_ref[1] + mp_ref[...]) + b_ref[...])
    mn_ref[...] = dinv * jnp.dot(h, w_ref[...])


def _final_body(s_ref, mp_ref, dinv_ref, b_ref, wc_ref, bc_ref,
                out_ref, h_ref):
    h = jnp.tanh(dinv_ref[...] * (s_ref[0] + s_ref[1] + mp_ref[...])
                 + b_ref[...])
    h_ref[...] = h
    out_ref[...] = jnp.dot(h, wc_ref[...]) + bc_ref[...]


def kernel(x, edge_index, W1, b1, W2, b2, W3, b3, Wc, bc):
    n, d_in = x.shape
    e = edge_index.shape[1]
    f1, f2, f3 = W1.shape[1], W2.shape[1], W3.shape[1]
    half = f1 // 2
    ncls = Wc.shape[1]
    br = 2000
    grid = (n // br,)
    nb = n // br

    ei3 = edge_index.reshape(2, e // CH, CH)
    ones16 = jnp.ones((CH, 16), jnp.float32)
    zeros = {f: jnp.zeros((n, f), jnp.float32) for f in {16, half, f2, f3}}

    degh = _make_deg_kernel(n, e)(ei3, ones16, zeros[16])

    rows = lambda i: (i, 0)
    rows_hi = lambda i: (i + nb, 0)
    fixed = lambda i: (0, 0)
    part = lambda i: (0, i, 0)

    dinv, mplo, mphi = pl.pallas_call(
        _prep_body,
        grid=grid,
        in_specs=[
            pl.BlockSpec((NC, br, 16), part),
            pl.BlockSpec((br, d_in), rows),
            pl.BlockSpec((d_in, f1), fixed),
        ],
        out_specs=[
            pl.BlockSpec((br, 1), rows),
            pl.BlockSpec((br, half), rows),
            pl.BlockSpec((br, half), rows),
        ],
        out_shape=[
            jax.ShapeDtypeStruct((n, 1), jnp.float32),
            jax.ShapeDtypeStruct((n, half), jnp.float32),
            jax.ShapeDtypeStruct((n, half), jnp.float32),
        ],
    )(degh, x, W1)

    s1f = _make_agg_split_kernel(n, e, half, 6, 3)(
        ei3, mplo, mphi, zeros[half])

    mp2 = pl.pallas_call(
        _mid_split_body,
        grid=grid,
        in_specs=[
            pl.BlockSpec((br, half), rows),
            pl.BlockSpec((br, half), rows_hi),
            pl.BlockSpec((br, half), rows),
            pl.BlockSpec((br, half), rows),
            pl.BlockSpec((br, 1), rows),
            pl.BlockSpec((1, f1), fixed),
            pl.BlockSpec((f1, f2), fixed),
        ],
        out_specs=pl.BlockSpec((br, f2), rows),
        out_shape=jax.ShapeDtypeStruct((n, f2), jnp.float32),
    )(s1f, s1f, mplo, mphi, dinv, b1.reshape(1, f1), W2)

    s2 = _make_agg_kernel(n, e, f2, 8, 4)(ei3, mp2, zeros[f2])

    mp3 = pl.pallas_call(
        _mid_body,
        grid=grid,
        in_specs=[
            pl.BlockSpec((NC, br, f2), part),
            pl.BlockSpec((br, f2), rows),
            pl.BlockSpec((br, 1), rows),
            pl.BlockSpec((1, f2), fixed),
            pl.BlockSpec((f2, f3), fixed),
        ],
        out_specs=pl.BlockSpec((br, f3), rows),
        out_shape=jax.ShapeDtypeStruct((n, f3), jnp.float32),
    )(s2, mp2, dinv, b2.reshape(1, f2), W3)

    s3 = _make_agg_kernel(n, e, f3, 16, 8)(ei3, mp3, zeros[f3])

    out, h3 = pl.pallas_call(
        _final_body,
        grid=grid,
        in_specs=[
            pl.BlockSpec((NC, br, f3), part),
            pl.BlockSpec((br, f3), rows),
            pl.BlockSpec((br, 1), rows),
            pl.BlockSpec((1, f3), fixed),
            pl.BlockSpec((f3, ncls), fixed),
            pl.BlockSpec((1, ncls), fixed),
        ],
        out_specs=[
            pl.BlockSpec((br, ncls), rows),
            pl.BlockSpec((br, f3), rows),
        ],
        out_shape=[
            jax.ShapeDtypeStruct((n, ncls), jnp.float32),
            jax.ShapeDtypeStruct((n, f3), jnp.float32),
        ],
    )(s3, mp3, dinv, b3.reshape(1, f3), Wc, bc.reshape(1, ncls))

    return (out, h3)
